# baseline (speedup 1.0000x reference)
import jax
import jax.numpy as jnp
from jax import lax
from jax.experimental import pallas as pl
from jax.experimental.pallas import tpu as pltpu
from jax.experimental.pallas import tpu_sc as plsc

B = 16
T = 16384
MAXSEQ = 2048
L = MAXSEQ + 2
L_PAD = 2176
CLS_ID = 2
SEP_ID = 3
LANES = 16
NCHUNK = (MAXSEQ + LANES) // LANES
WIN = 2080
WIN_BUF = 4176


def _row_body(tokens_hbm, splits_hbm, out_hbm, splits_v, win_v, row_v, sem1, sem2):
    c = lax.axis_index("c")
    s = lax.axis_index("s")
    wid = s + c * 0

    @pl.when(wid < B)
    def _():
        cp1 = pltpu.async_copy(splits_hbm, splits_v.at[pl.ds(0, B + 1)], sem1)
        cp2 = pltpu.async_copy(tokens_hbm, win_v.at[pl.ds(0, T)], sem2)
        lane = lax.broadcasted_iota(jnp.int32, (LANES,), 0)
        cp1.wait()
        cp2.wait()
        start = splits_v[pl.ds(wid, LANES)][0]
        end = splits_v[pl.ds(wid + 1, LANES)][0]
        ln = end - start

        @plsc.parallel_loop(0, NCHUNK * LANES, step=LANES, unroll=4)
        def chunk(i):
            idx = lane + i
            tok = win_v[pl.ds(start + i, LANES)]
            val = jnp.where(idx < ln, tok,
                            jnp.where(idx == ln,
                                      jnp.full((LANES,), SEP_ID, jnp.int32),
                                      jnp.zeros((LANES,), jnp.int32)))
            row_v[pl.ds(1 + i, LANES)] = val
        head = row_v[pl.ds(0, LANES)]
        row_v[pl.ds(0, LANES)] = jnp.where(
            lane == 0, jnp.full((LANES,), CLS_ID, jnp.int32), head)
        pltpu.sync_copy(row_v.at[pl.ds(0, L_PAD)], out_hbm.at[wid])


def kernel(token_ids, row_splits):
    mesh = plsc.VectorSubcoreMesh(core_axis_name="c", subcore_axis_name="s", num_cores=1)
    f = pl.kernel(
        _row_body,
        out_type=jax.ShapeDtypeStruct((B, L_PAD), jnp.int32),
        mesh=mesh,
        scratch_types=[
            pltpu.VMEM((3 * B,), jnp.int32),
            pltpu.VMEM((T + WIN,), jnp.int32),
            pltpu.VMEM((L_PAD,), jnp.int32),
            pltpu.SemaphoreType.DMA,
            pltpu.SemaphoreType.DMA,
        ],
    )
    return f(token_ids, row_splits)[:, :L]


# speedup vs baseline: 1.0608x; 1.0608x over previous
import jax
import jax.numpy as jnp
from jax import lax
from jax.experimental import pallas as pl
from jax.experimental.pallas import tpu as pltpu
from jax.experimental.pallas import tpu_sc as plsc

B = 16
T = 16384
MAXSEQ = 2048
L = MAXSEQ + 2
L_PAD = 2176
CLS_ID = 2
SEP_ID = 3
LANES = 16
NCHUNK = (MAXSEQ + LANES) // LANES
WIN = 2080
WIN_BUF = 4176


def _row_body(tokens_hbm, splits_hbm, out_hbm, splits_v, win_v, row_v):
    c = lax.axis_index("c")
    s = lax.axis_index("s")
    wid = s + c * 0

    @pl.when(wid < B)
    def _():
        pltpu.sync_copy(splits_hbm, splits_v.at[pl.ds(0, B + 1)])
        lane = lax.broadcasted_iota(jnp.int32, (LANES,), 0)
        start = splits_v[pl.ds(wid, LANES)][0]
        end = splits_v[pl.ds(wid + 1, LANES)][0]
        ln = end - start
        start_al = jnp.minimum((start // 8) * 8, T - WIN)
        off = start - start_al
        pltpu.sync_copy(tokens_hbm.at[pl.ds(start_al, WIN)], win_v.at[pl.ds(0, WIN)])

        @plsc.parallel_loop(0, NCHUNK * LANES, step=LANES, unroll=4)
        def chunk(i):
            idx = lane + i
            tok = win_v[pl.ds(off + i, LANES)]
            val = jnp.where(idx < ln, tok,
                            jnp.where(idx == ln,
                                      jnp.full((LANES,), SEP_ID, jnp.int32),
                                      jnp.zeros((LANES,), jnp.int32)))
            row_v[pl.ds(1 + i, LANES)] = val
        head = row_v[pl.ds(0, LANES)]
        row_v[pl.ds(0, LANES)] = jnp.where(
            lane == 0, jnp.full((LANES,), CLS_ID, jnp.int32), head)
        pltpu.sync_copy(row_v.at[pl.ds(0, L_PAD)], out_hbm.at[wid])


def kernel(token_ids, row_splits):
    mesh = plsc.VectorSubcoreMesh(core_axis_name="c", subcore_axis_name="s", num_cores=1)
    f = pl.kernel(
        _row_body,
        out_type=jax.ShapeDtypeStruct((B, L_PAD), jnp.int32),
        mesh=mesh,
        scratch_types=[
            pltpu.VMEM((3 * B,), jnp.int32),
            pltpu.VMEM((WIN_BUF,), jnp.int32),
            pltpu.VMEM((L_PAD,), jnp.int32),
        ],
    )
    return f(token_ids, row_splits)[:, :L]
